# baseline (speedup 1.0000x reference)
import functools

import jax
import jax.numpy as jnp
from jax import lax
from jax.experimental import pallas as pl
from jax.experimental.pallas import tpu as pltpu
from jax.experimental.pallas import tpu_sc as plsc

N_WORDS = 1_000_000
N_LABELS = 64
VBLK = 16_384
GRID = -(-N_WORDS // VBLK)

TOKENS = 4096 * 200
NUM_WORKERS = 32
PER_W = TOKENS // NUM_WORKERS
CHUNK = 128
NCHUNK = PER_W // CHUNK
LANES = 16


def _row_log_softmax(v):
    m = jnp.max(v, axis=-1, keepdims=True)
    return v - (m + jnp.log(jnp.sum(jnp.exp(v - m), axis=-1, keepdims=True)))


def _reduce_body(emits_t_ref, trans_ref, start_ref, end_ref,
                 lse_ref, pairs_ref, trans_out, start_out, end_out,
                 m_scr, s_scr):
    i = pl.program_id(0)

    @pl.when(i == 0)
    def _():
        m_scr[...] = jnp.full(m_scr.shape, -jnp.inf, m_scr.dtype)
        s_scr[...] = jnp.zeros(s_scr.shape, s_scr.dtype)

    x = emits_t_ref[...]
    lane = jax.lax.broadcasted_iota(jnp.int32, x.shape, 1) + i * VBLK
    valid = lane < N_WORDS
    xm = jnp.where(valid, x, -jnp.inf)
    bm = jnp.max(xm, axis=1, keepdims=True)
    m_old = m_scr[...]
    m_new = jnp.maximum(m_old, bm)
    e = jnp.where(valid, jnp.exp(x - m_new), 0.0)
    s_new = s_scr[...] * jnp.exp(m_old - m_new) + jnp.sum(e, axis=1,
                                                          keepdims=True)
    m_scr[...] = m_new
    s_scr[...] = s_new

    t = jnp.transpose(x)
    pairs_ref[...] = jnp.concatenate([t, jnp.zeros_like(t)], axis=1)

    @pl.when(i == GRID - 1)
    def _():
        lse_ref[...] = m_new + jnp.log(s_new)
        trans_out[...] = _row_log_softmax(trans_ref[...])
        start_out[...] = _row_log_softmax(start_ref[...])
        end_out[...] = _row_log_softmax(end_ref[...])


def _lse_table_and_small(emits_t, transitions, start, end):
    out_shapes = (
        jax.ShapeDtypeStruct((N_LABELS, 1), jnp.float32),
        jax.ShapeDtypeStruct((N_WORDS, 128), jnp.float32),
        jax.ShapeDtypeStruct((N_LABELS, N_LABELS), jnp.float32),
        jax.ShapeDtypeStruct((1, N_LABELS), jnp.float32),
        jax.ShapeDtypeStruct((1, N_LABELS), jnp.float32),
    )
    return pl.pallas_call(
        _reduce_body,
        grid=(GRID,),
        in_specs=[
            pl.BlockSpec((N_LABELS, VBLK), lambda i: (0, i)),
            pl.BlockSpec((N_LABELS, N_LABELS), lambda i: (0, 0)),
            pl.BlockSpec((1, N_LABELS), lambda i: (0, 0)),
            pl.BlockSpec((1, N_LABELS), lambda i: (0, 0)),
        ],
        out_specs=(
            pl.BlockSpec((N_LABELS, 1), lambda i: (0, 0)),
            pl.BlockSpec((VBLK, 128), lambda i: (i, 0)),
            pl.BlockSpec((N_LABELS, N_LABELS), lambda i: (0, 0)),
            pl.BlockSpec((1, N_LABELS), lambda i: (0, 0)),
            pl.BlockSpec((1, N_LABELS), lambda i: (0, 0)),
        ),
        out_shape=out_shapes,
        scratch_shapes=[
            pltpu.VMEM((N_LABELS, 1), jnp.float32),
            pltpu.VMEM((N_LABELS, 1), jnp.float32),
        ],
    )(emits_t, transitions, start.reshape(1, N_LABELS),
      end.reshape(1, N_LABELS))


def _make_gather():
    mesh = plsc.VectorSubcoreMesh(core_axis_name="c", subcore_axis_name="s")

    @functools.partial(
        pl.kernel,
        mesh=mesh,
        compiler_params=pltpu.CompilerParams(use_tc_tiling_on_sc=False),
        out_type=jax.ShapeDtypeStruct((TOKENS, 128), jnp.float32),
        scratch_types=[
            pltpu.VMEM((NCHUNK, CHUNK), jnp.int32),
            pltpu.VMEM((CHUNK, N_LABELS), jnp.float32),
            pltpu.VMEM((CHUNK, N_LABELS), jnp.float32),
            pltpu.VMEM((CHUNK, N_LABELS), jnp.float32),
            pltpu.VMEM((CHUNK, N_LABELS), jnp.float32),
            pltpu.VMEM((N_LABELS,), jnp.float32),
            pltpu.SemaphoreType.DMA,
            pltpu.SemaphoreType.DMA,
            pltpu.SemaphoreType.DMA,
            pltpu.SemaphoreType.DMA,
            pltpu.SemaphoreType.DMA,
            pltpu.SemaphoreType.DMA,
            pltpu.SemaphoreType.DMA,
            pltpu.SemaphoreType.DMA,
        ],
    )
    def gather_kernel(words_hbm, emits_hbm, lse_hbm, out_hbm,
                      idx_v, r0, r1, r2, r3, lse_v,
                      g0, g1, g2, g3, o0, o1, o2, o3):
        wid = lax.axis_index("s") * 2 + lax.axis_index("c")
        pltpu.sync_copy(words_hbm.at[pl.ds(wid * NCHUNK, NCHUNK)], idx_v)
        pltpu.sync_copy(lse_hbm, lse_v)
        base = wid * PER_W
        bufs = (r0, r1, r2, r3)
        gsems = (g0, g1, g2, g3)
        osems = (o0, o1, o2, o3)
        NB = 4

        def out_slice(g):
            return out_hbm.at[pl.ds(base + g * CHUNK, CHUNK),
                              pl.ds(0, N_LABELS)]

        lse_regs = [lse_v[pl.ds(c * LANES, LANES)]
                    for c in range(N_LABELS // LANES)]

        def fix_rows(rows):
            def row_body(r, c2):
                for c in range(N_LABELS // LANES):
                    sl = pl.ds(c * LANES, LANES)
                    rows[r, sl] = rows[r, sl] - lse_regs[c]
                return c2
            lax.fori_loop(0, CHUNK, row_body, 0, unroll=8)

        pltpu.async_copy(emits_hbm.at[idx_v.at[0]], bufs[0], gsems[0])
        pltpu.async_copy(emits_hbm.at[idx_v.at[1]], bufs[1], gsems[1])

        def chunk_quad(p, carry):
            for b in range(NB):
                g = p * NB + b
                nb = (b + 2) % NB

                @pl.when(g - 2 >= 0)
                def _():
                    pltpu.make_async_copy(
                        bufs[nb], out_slice(g - 2), osems[nb]).wait()

                @pl.when(g + 2 < NCHUNK)
                def _():
                    pltpu.async_copy(
                        emits_hbm.at[idx_v.at[g + 2]], bufs[nb], gsems[nb])

                pltpu.make_async_copy(
                    emits_hbm.at[idx_v.at[g]], bufs[b], gsems[b]).wait()
                fix_rows(bufs[b])
                pltpu.async_copy(bufs[b], out_slice(g), osems[b])
            return carry

        lax.fori_loop(0, NCHUNK // NB, chunk_quad, 0)

        pltpu.make_async_copy(
            bufs[(NCHUNK - 2) % NB], out_slice(NCHUNK - 2),
            osems[(NCHUNK - 2) % NB]).wait()
        pltpu.make_async_copy(
            bufs[(NCHUNK - 1) % NB], out_slice(NCHUNK - 1),
            osems[(NCHUNK - 1) % NB]).wait()

    return gather_kernel


def _make_gather_fmt():
    mesh = plsc.VectorSubcoreMesh(core_axis_name="c", subcore_axis_name="s")
    SEQ = 200
    NBT = 4096 // 128

    @functools.partial(
        pl.kernel,
        mesh=mesh,
        compiler_params=pltpu.CompilerParams(use_tc_tiling_on_sc=False,
                                             needs_layout_passes=False),
        out_type=jax.ShapeDtypeStruct((SEQ, 8, NBT, 8, 128), jnp.float32),
        scratch_types=[
            pltpu.VMEM((SEQ, CHUNK), jnp.int32),
            pltpu.VMEM((CHUNK, N_LABELS), jnp.float32),
            pltpu.VMEM((CHUNK, N_LABELS), jnp.float32),
            pltpu.VMEM((8, 8, 129), jnp.float32),
            pltpu.VMEM((8, 8, 129), jnp.float32),
            pltpu.VMEM((N_LABELS,), jnp.float32),
            pltpu.SemaphoreType.DMA,
            pltpu.SemaphoreType.DMA,
            pltpu.SemaphoreType.DMA,
            pltpu.SemaphoreType.DMA,
        ],
    )
    def gather_kernel(words_hbm, emits_hbm, lse_hbm, out_hbm,
                      idx_v, r0, r1, t0, t1, lse_v, g0, g1, o0, o1):
        wid = lax.axis_index("s") * 2 + lax.axis_index("c")
        pltpu.sync_copy(words_hbm.at[:, pl.ds(wid * CHUNK, CHUNK)], idx_v)
        pltpu.sync_copy(lse_hbm, lse_v)
        rows = (r0, r1)
        tbufs = (t0, t1)
        gsems = (g0, g1)
        osems = (o0, o1)

        lse_regs = [lse_v[pl.ds(c * LANES, LANES)]
                    for c in range(N_LABELS // LANES)]
        iota16 = lax.iota(jnp.int32, LANES)
        lt_idx = [(iota16 + c * LANES) // 8 for c in range(N_LABELS // LANES)]
        li_idx = [(iota16 + c * LANES) % 8 for c in range(N_LABELS // LANES)]

        def transpose_tile(rb, tb):
            def tok_body(t, c2):
                bi = jnp.full((LANES,), 0, jnp.int32) + t
                for c in range(N_LABELS // LANES):
                    v = rb[t, pl.ds(c * LANES, LANES)] - lse_regs[c]
                    plsc.store_scatter(tb, [lt_idx[c], li_idx[c], bi], v)
                return c2
            lax.fori_loop(0, CHUNK, tok_body, 0, unroll=8)

        pltpu.async_copy(emits_hbm.at[idx_v.at[0]], rows[0], gsems[0])

        def s_body(p, carry):
            for b in range(2):
                s = p * 2 + b
                pltpu.make_async_copy(
                    emits_hbm.at[idx_v.at[s]], rows[b], gsems[b]).wait()

                @pl.when(s + 1 < SEQ)
                def _():
                    pltpu.async_copy(
                        emits_hbm.at[idx_v.at[s + 1]], rows[1 - b],
                        gsems[1 - b])

                @pl.when(s - 2 >= 0)
                def _():
                    pltpu.make_async_copy(
                        tbufs[b].at[:, :, pl.ds(0, 128)],
                        out_hbm.at[s - 2, :, wid], osems[b]).wait()

                transpose_tile(rows[b], tbufs[b])
                pltpu.async_copy(tbufs[b].at[:, :, pl.ds(0, 128)],
                                 out_hbm.at[s, :, wid], osems[b])
            return carry

        lax.fori_loop(0, SEQ // 2, s_body, 0)
        pltpu.make_async_copy(
            tbufs[0].at[:, :, pl.ds(0, 128)],
            out_hbm.at[SEQ - 2, :, wid], osems[0]).wait()
        pltpu.make_async_copy(
            tbufs[1].at[:, :, pl.ds(0, 128)],
            out_hbm.at[SEQ - 1, :, wid], osems[1]).wait()

    return gather_kernel


def kernel(words, mask, emits, transitions, start, end):
    lse2, table128, trans_ls, start_ls2, end_ls2 = _lse_table_and_small(
        emits.T, transitions, start, end)
    table2 = table128.reshape(2 * N_WORDS, N_LABELS)
    words_t = (words * 2).T
    out5 = _make_gather_fmt()(words_t, table2, lse2.reshape(N_LABELS))
    emit_scores = out5.transpose(2, 4, 0, 1, 3).reshape(
        words.shape[0], words.shape[1], N_LABELS)
    return (emit_scores, trans_ls, start_ls2.reshape(N_LABELS),
            end_ls2.reshape(N_LABELS))


# speedup vs baseline: 1.2784x; 1.2784x over previous
import functools

import jax
import jax.numpy as jnp
from jax import lax
from jax.experimental import pallas as pl
from jax.experimental.pallas import tpu as pltpu
from jax.experimental.pallas import tpu_sc as plsc

N_WORDS = 1_000_000
N_LABELS = 64
VBLK = 16_384
GRID = -(-N_WORDS // VBLK)

TOKENS = 4096 * 200
NUM_WORKERS = 32
PER_W = TOKENS // NUM_WORKERS
CHUNK = 128
NCHUNK = PER_W // CHUNK
LANES = 16


def _row_log_softmax(v):
    m = jnp.max(v, axis=-1, keepdims=True)
    return v - (m + jnp.log(jnp.sum(jnp.exp(v - m), axis=-1, keepdims=True)))


def _reduce_body(emits_t_ref, trans_ref, start_ref, end_ref,
                 lse_ref, pairs_ref, trans_out, start_out, end_out,
                 m_scr, s_scr):
    i = pl.program_id(0)

    @pl.when(i == 0)
    def _():
        m_scr[...] = jnp.full(m_scr.shape, -jnp.inf, m_scr.dtype)
        s_scr[...] = jnp.zeros(s_scr.shape, s_scr.dtype)

    x = emits_t_ref[...]
    lane = jax.lax.broadcasted_iota(jnp.int32, x.shape, 1) + i * VBLK
    valid = lane < N_WORDS
    xm = jnp.where(valid, x, -jnp.inf)
    bm = jnp.max(xm, axis=1, keepdims=True)
    m_old = m_scr[...]
    m_new = jnp.maximum(m_old, bm)
    e = jnp.where(valid, jnp.exp(x - m_new), 0.0)
    s_new = s_scr[...] * jnp.exp(m_old - m_new) + jnp.sum(e, axis=1,
                                                          keepdims=True)
    m_scr[...] = m_new
    s_scr[...] = s_new

    t = jnp.transpose(x)
    pairs_ref[...] = jnp.concatenate([t, jnp.zeros_like(t)], axis=1)

    @pl.when(i == GRID - 1)
    def _():
        lse_ref[...] = m_new + jnp.log(s_new)
        trans_out[...] = _row_log_softmax(trans_ref[...])
        start_out[...] = _row_log_softmax(start_ref[...])
        end_out[...] = _row_log_softmax(end_ref[...])


def _lse_table_and_small(emits_t, transitions, start, end):
    out_shapes = (
        jax.ShapeDtypeStruct((N_LABELS, 1), jnp.float32),
        jax.ShapeDtypeStruct((N_WORDS, 128), jnp.float32),
        jax.ShapeDtypeStruct((N_LABELS, N_LABELS), jnp.float32),
        jax.ShapeDtypeStruct((1, N_LABELS), jnp.float32),
        jax.ShapeDtypeStruct((1, N_LABELS), jnp.float32),
    )
    return pl.pallas_call(
        _reduce_body,
        grid=(GRID,),
        in_specs=[
            pl.BlockSpec((N_LABELS, VBLK), lambda i: (0, i)),
            pl.BlockSpec((N_LABELS, N_LABELS), lambda i: (0, 0)),
            pl.BlockSpec((1, N_LABELS), lambda i: (0, 0)),
            pl.BlockSpec((1, N_LABELS), lambda i: (0, 0)),
        ],
        out_specs=(
            pl.BlockSpec((N_LABELS, 1), lambda i: (0, 0)),
            pl.BlockSpec((VBLK, 128), lambda i: (i, 0)),
            pl.BlockSpec((N_LABELS, N_LABELS), lambda i: (0, 0)),
            pl.BlockSpec((1, N_LABELS), lambda i: (0, 0)),
            pl.BlockSpec((1, N_LABELS), lambda i: (0, 0)),
        ),
        out_shape=out_shapes,
        scratch_shapes=[
            pltpu.VMEM((N_LABELS, 1), jnp.float32),
            pltpu.VMEM((N_LABELS, 1), jnp.float32),
        ],
    )(emits_t, transitions, start.reshape(1, N_LABELS),
      end.reshape(1, N_LABELS))


def _make_gather():
    mesh = plsc.VectorSubcoreMesh(core_axis_name="c", subcore_axis_name="s")

    @functools.partial(
        pl.kernel,
        mesh=mesh,
        compiler_params=pltpu.CompilerParams(use_tc_tiling_on_sc=False),
        out_type=jax.ShapeDtypeStruct((TOKENS, 128), jnp.float32),
        scratch_types=[
            pltpu.VMEM((NCHUNK, CHUNK), jnp.int32),
            pltpu.VMEM((CHUNK, N_LABELS), jnp.float32),
            pltpu.VMEM((CHUNK, N_LABELS), jnp.float32),
            pltpu.VMEM((CHUNK, N_LABELS), jnp.float32),
            pltpu.VMEM((CHUNK, N_LABELS), jnp.float32),
            pltpu.VMEM((N_LABELS,), jnp.float32),
            pltpu.SemaphoreType.DMA,
            pltpu.SemaphoreType.DMA,
            pltpu.SemaphoreType.DMA,
            pltpu.SemaphoreType.DMA,
            pltpu.SemaphoreType.DMA,
            pltpu.SemaphoreType.DMA,
            pltpu.SemaphoreType.DMA,
            pltpu.SemaphoreType.DMA,
        ],
    )
    def gather_kernel(words_hbm, emits_hbm, lse_hbm, out_hbm,
                      idx_v, r0, r1, r2, r3, lse_v,
                      g0, g1, g2, g3, o0, o1, o2, o3):
        wid = lax.axis_index("s") * 2 + lax.axis_index("c")
        pltpu.sync_copy(words_hbm.at[pl.ds(wid * NCHUNK, NCHUNK)], idx_v)
        pltpu.sync_copy(lse_hbm, lse_v)
        base = wid * PER_W
        bufs = (r0, r1, r2, r3)
        gsems = (g0, g1, g2, g3)
        osems = (o0, o1, o2, o3)
        NB = 4

        def out_slice(g):
            return out_hbm.at[pl.ds(base + g * CHUNK, CHUNK),
                              pl.ds(0, N_LABELS)]

        lse_regs = [lse_v[pl.ds(c * LANES, LANES)]
                    for c in range(N_LABELS // LANES)]

        def fix_rows(rows):
            def row_body(r, c2):
                for c in range(N_LABELS // LANES):
                    sl = pl.ds(c * LANES, LANES)
                    rows[r, sl] = rows[r, sl] - lse_regs[c]
                return c2
            lax.fori_loop(0, CHUNK, row_body, 0, unroll=8)

        pltpu.async_copy(emits_hbm.at[idx_v.at[0]], bufs[0], gsems[0])
        pltpu.async_copy(emits_hbm.at[idx_v.at[1]], bufs[1], gsems[1])

        def chunk_quad(p, carry):
            for b in range(NB):
                g = p * NB + b
                nb = (b + 2) % NB

                @pl.when(g - 2 >= 0)
                def _():
                    pltpu.make_async_copy(
                        bufs[nb], out_slice(g - 2), osems[nb]).wait()

                @pl.when(g + 2 < NCHUNK)
                def _():
                    pltpu.async_copy(
                        emits_hbm.at[idx_v.at[g + 2]], bufs[nb], gsems[nb])

                pltpu.make_async_copy(
                    emits_hbm.at[idx_v.at[g]], bufs[b], gsems[b]).wait()
                fix_rows(bufs[b])
                pltpu.async_copy(bufs[b], out_slice(g), osems[b])
            return carry

        lax.fori_loop(0, NCHUNK // NB, chunk_quad, 0)

        pltpu.make_async_copy(
            bufs[(NCHUNK - 2) % NB], out_slice(NCHUNK - 2),
            osems[(NCHUNK - 2) % NB]).wait()
        pltpu.make_async_copy(
            bufs[(NCHUNK - 1) % NB], out_slice(NCHUNK - 1),
            osems[(NCHUNK - 1) % NB]).wait()

    return gather_kernel


def kernel(words, mask, emits, transitions, start, end):
    lse2, table128, trans_ls, start_ls2, end_ls2 = _lse_table_and_small(
        emits.T, transitions, start, end)
    table2 = table128.reshape(2 * N_WORDS, N_LABELS)
    words2d = (words * 2).reshape(TOKENS // CHUNK, CHUNK)
    out = _make_gather()(words2d, table2, lse2.reshape(N_LABELS))
    emit_scores = out.reshape(words.shape[0], words.shape[1], 128)[:, :, :N_LABELS]
    return (emit_scores, trans_ls, start_ls2.reshape(N_LABELS),
            end_ls2.reshape(N_LABELS))
